# Initial kernel scaffold; baseline (speedup 1.0000x reference)
import jax
import jax.numpy as jnp
from jax.experimental import pallas as pl


def kernel(x, pos, edge_index, batch, params):
    raise NotImplementedError("write your pallas kernel here")


# speedup vs baseline: 1.5476x; 1.5476x over previous
import functools

import jax
import jax.numpy as jnp
from jax import lax
from jax.experimental import pallas as pl
from jax.experimental.pallas import tpu as pltpu
from jax.experimental.pallas import tpu_sc as plsc

_NC, _NS = 2, 16
_NW = _NC * _NS
_C = 80
_EPS = 1e-5


def _sc_mesh():
    return plsc.VectorSubcoreMesh(
        core_axis_name="c", subcore_axis_name="s",
        num_cores=_NC, num_subcores=_NS)


_SC_PARAMS = pltpu.CompilerParams(use_tc_tiling_on_sc=False)


def _gather2(tab, src, dst):
    E = src.shape[0]
    w = tab.shape[1]
    epw = E // _NW
    nch = epw // _C

    @functools.partial(
        pl.kernel,
        out_type=(jax.ShapeDtypeStruct((E, w), jnp.float32),
                  jax.ShapeDtypeStruct((E, w), jnp.float32)),
        mesh=_sc_mesh(),
        compiler_params=_SC_PARAMS,
        scratch_types=[
            pltpu.VMEM((_C,), jnp.int32),
            pltpu.VMEM((_C,), jnp.int32),
            pltpu.VMEM((_C, w), jnp.float32),
            pltpu.VMEM((_C, w), jnp.float32),
            pltpu.SemaphoreType.DMA,
            pltpu.SemaphoreType.DMA,
        ])
    def k(t_h, src_h, dst_h, xi_h, xj_h, idx_d, idx_s, bi, bj, sm1, sm2):
        wid = lax.axis_index("s") * _NC + lax.axis_index("c")
        base0 = wid * epw

        def chunk(j, carry):
            base = base0 + j * _C
            pltpu.sync_copy(dst_h.at[pl.ds(base, _C)], idx_d)
            pltpu.sync_copy(src_h.at[pl.ds(base, _C)], idx_s)
            ci = pltpu.async_copy(t_h.at[idx_d], bi, sm1)
            cj = pltpu.async_copy(t_h.at[idx_s], bj, sm2)
            ci.wait()
            cj.wait()
            pltpu.sync_copy(bi, xi_h.at[pl.ds(base, _C)])
            pltpu.sync_copy(bj, xj_h.at[pl.ds(base, _C)])
            return carry

        lax.fori_loop(0, nch, chunk, 0)

    return k(tab, src, dst)


def _seg_sum(dst, n, w, vals=None, table=None, src=None):
    E = dst.shape[0]
    epw = E // _NW
    nch = epw // _C
    rps = n // _NS
    zr = 125
    nz = rps // zr
    mode = 'linear' if vals is not None else ('gather' if table is not None
                                              else 'ones')

    scratch = [
        pltpu.VMEM((_C,), jnp.int32),
        pltpu.VMEM((_C,), jnp.int32),
        pltpu.VMEM((_C, w), jnp.float32),
        pltpu.VMEM((zr, w), jnp.float32),
        pltpu.VMEM_SHARED((n, w), jnp.float32),
        pltpu.SemaphoreType.DMA,
    ]

    def body(refs):
        (v_h, dst_h, src_h, out_h, idx_d, idx_s, vb, zb, acc, sem) = refs
        sid = lax.axis_index("s")
        cid = lax.axis_index("c")

        def zrow(i, c):
            for h in range(w // 16):
                zb[i, pl.ds(16 * h, 16)] = jnp.zeros((16,), jnp.float32)
            return c

        lax.fori_loop(0, zr, zrow, 0)
        for t in range(nz):
            pltpu.sync_copy(zb, acc.at[pl.ds(sid * rps + t * zr, zr)])
        if mode == 'ones':
            def orow(i, c):
                for h in range(w // 16):
                    vb[i, pl.ds(16 * h, 16)] = jnp.ones((16,), jnp.float32)
                return c
            lax.fori_loop(0, _C, orow, 0)
        plsc.subcore_barrier()

        wid = sid * _NC + cid
        base0 = wid * epw

        def chunk(j, carry):
            base = base0 + j * _C
            pltpu.sync_copy(dst_h.at[pl.ds(base, _C)], idx_d)
            if mode == 'linear':
                pltpu.sync_copy(v_h.at[pl.ds(base, _C)], vb)
            elif mode == 'gather':
                pltpu.sync_copy(src_h.at[pl.ds(base, _C)], idx_s)
                pltpu.async_copy(v_h.at[idx_s], vb, sem).wait()
            pltpu.sync_copy(vb, acc.at[idx_d], add=True)
            return carry

        lax.fori_loop(0, nch, chunk, 0)
        plsc.subcore_barrier()
        for t in range(nz):
            row0 = sid * rps + t * zr
            pltpu.sync_copy(acc.at[pl.ds(row0, zr)], zb)
            pltpu.sync_copy(zb, out_h.at[pl.ds(cid * n + row0, zr)])

    kern = functools.partial(
        pl.kernel,
        out_type=jax.ShapeDtypeStruct((_NC * n, w), jnp.float32),
        mesh=_sc_mesh(),
        compiler_params=_SC_PARAMS,
        scratch_types=scratch)

    if mode == 'linear':
        @kern
        def k(v_h, dst_h, out_h, *s):
            body((v_h, dst_h, None) + (out_h,) + s)
        return k(vals, dst)
    elif mode == 'gather':
        @kern
        def k(v_h, dst_h, src_h, out_h, *s):
            body((v_h, dst_h, src_h) + (out_h,) + s)
        return k(table, dst, src)
    else:
        @kern
        def k(dst_h, out_h, *s):
            body((None, dst_h, None) + (out_h,) + s)
        return k(dst)


_BE = 2000


def _edge_z(XI, XJ, ci, W1, b1):
    E = XI.shape[0]
    wp = XI.shape[1]
    g = E // _BE

    def body(xi_ref, xj_ref, w_ref, b_ref, h_ref):
        xi = xi_ref[...][:, :ci]
        xj = xj_ref[...][:, :ci]
        t = jnp.concatenate([xi, xj - xi], axis=1)
        z = jnp.dot(t, w_ref[...],
                    preferred_element_type=jnp.float32) + b_ref[...]
        h_ref[...] = jnp.tanh(z)

    return pl.pallas_call(
        body,
        grid=(g,),
        in_specs=[pl.BlockSpec((_BE, wp), lambda i: (i, 0)),
                  pl.BlockSpec((_BE, wp), lambda i: (i, 0)),
                  pl.BlockSpec((2 * ci, 32), lambda i: (0, 0)),
                  pl.BlockSpec((1, 32), lambda i: (0, 0))],
        out_specs=pl.BlockSpec((_BE, 32), lambda i: (i, 0)),
        out_shape=jax.ShapeDtypeStruct((E, 32), jnp.float32),
    )(XI, XJ, W1, b1)


def _edge_mm(Hn, W, b, tanh):
    E = Hn.shape[0]
    wp = W.shape[1]
    g = E // _BE

    def body(h_ref, w_ref, b_ref, o_ref):
        z = jnp.dot(h_ref[...], w_ref[...],
                    preferred_element_type=jnp.float32) + b_ref[...]
        o_ref[...] = jnp.tanh(z) if tanh else z

    return pl.pallas_call(
        body,
        grid=(g,),
        in_specs=[pl.BlockSpec((_BE, 32), lambda i: (i, 0)),
                  pl.BlockSpec((32, wp), lambda i: (0, 0)),
                  pl.BlockSpec((1, wp), lambda i: (0, 0))],
        out_specs=pl.BlockSpec((_BE, wp), lambda i: (i, 0)),
        out_shape=jax.ShapeDtypeStruct((E, wp), jnp.float32),
    )(Hn, W, b)


def _node(X1, W1, X2=None, W2=None, b=None, T=None, leaky=False):
    n = X1.shape[0]
    co = W1.shape[1]
    bn = 2000
    g = n // bn

    has2 = X2 is not None
    hasT = T is not None
    hasb = b is not None

    def body(*refs):
        it = iter(refs)
        x1 = next(it)[...]
        w1 = next(it)[...]
        acc = jnp.dot(x1, w1, preferred_element_type=jnp.float32)
        if has2:
            acc += jnp.dot(next(it)[...], next(it)[...],
                           preferred_element_type=jnp.float32)
        if hasT:
            acc += next(it)[...]
        if hasb:
            acc += next(it)[...]
        o = next(it)
        if leaky:
            acc = jnp.where(acc >= 0, acc, 0.1 * acc)
        o[...] = acc

    c1 = X1.shape[1]
    in_specs = [pl.BlockSpec((bn, c1), lambda i: (i, 0)),
                pl.BlockSpec((c1, co), lambda i: (0, 0))]
    args = [X1, W1]
    if has2:
        c2 = X2.shape[1]
        in_specs += [pl.BlockSpec((bn, c2), lambda i: (i, 0)),
                     pl.BlockSpec((c2, co), lambda i: (0, 0))]
        args += [X2, W2]
    if hasT:
        in_specs.append(pl.BlockSpec((bn, co), lambda i: (i, 0)))
        args.append(T)
    if hasb:
        in_specs.append(pl.BlockSpec((1, co), lambda i: (0, 0)))
        args.append(b)

    return pl.pallas_call(
        body,
        grid=(g,),
        in_specs=in_specs,
        out_specs=pl.BlockSpec((bn, co), lambda i: (i, 0)),
        out_shape=jax.ShapeDtypeStruct((n, co), jnp.float32),
    )(*args)


def _padw(a, w):
    return a if a.shape[1] == w else jnp.pad(a, ((0, 0), (0, w - a.shape[1])))


def kernel(x, pos, edge_index, batch, params):
    src = edge_index[0].astype(jnp.int32)
    dst = edge_index[1].astype(jnp.int32)
    n = x.shape[0]
    E = src.shape[0]

    cntP = _seg_sum(dst, n, 16)
    cnt = cntP[:n, 0] + cntP[n:, 0]
    cmax = jnp.maximum(cnt, 1.0)

    def _bn(h, g, b):
        m = jnp.mean(h, axis=0)
        v = jnp.var(h, axis=0)
        return (h - m) / jnp.sqrt(v + _EPS) * g + b

    def ec_layer(p, xin):
        ci = xin.shape[1]
        wp = max(16, ci)
        XI, XJ = _gather2(_padw(xin, wp), src, dst)
        H1 = _edge_z(XI, XJ, ci, p['W1'], p['b1'][None])
        H1n = _bn(H1, p['g1'], p['be1'])
        H2 = _edge_mm(H1n, p['W2'], p['b2'][None], tanh=True)
        H2n = _bn(H2, p['g2'], p['be2'])
        cop = max(16, p['W3'].shape[1])
        Y = _edge_mm(H2n, _padw(p['W3'], cop), _padw(p['b3'][None], cop),
                     tanh=False)
        P = _seg_sum(dst, n, cop, vals=Y)
        return (P[:n] + P[n:])[:, :p['W3'].shape[1]]

    h = ec_layer(params['ec1'], pos)
    app = h
    for nm in ['eca', 'ecb', 'ecc']:
        h = ec_layer(params[nm], h)
        app = jnp.concatenate([app, h], axis=1)
    h = ec_layer(params['ec2'], app)
    h = ec_layer(params['ec3'], h)

    def sage(p, u, leaky):
        ci = u.shape[1]
        w = max(16, ci)
        S = _seg_sum(dst, n, w, table=_padw(u, w), src=src)
        mean = (S[:n, :ci] + S[n:, :ci]) / cmax[:, None]
        return _node(mean, p['Wl'], X2=u, W2=p['Wr'], b=p['b'][None],
                     leaky=leaky)

    u = sage(params['sg4'], x + h, True)
    for nm in ['sga', 'sgb', 'sgc']:
        u = sage(params[nm], u, True)
    return sage(params['sg5'], u, False)


# speedup vs baseline: 1.9217x; 1.2417x over previous
import functools

import jax
import jax.numpy as jnp
from jax import lax
from jax.experimental import pallas as pl
from jax.experimental.pallas import tpu as pltpu
from jax.experimental.pallas import tpu_sc as plsc

_NC, _NS = 2, 16
_NW = _NC * _NS
_C = 80
_EPS = 1e-5


def _sc_mesh():
    return plsc.VectorSubcoreMesh(
        core_axis_name="c", subcore_axis_name="s",
        num_cores=_NC, num_subcores=_NS)


_SC_PARAMS = pltpu.CompilerParams(use_tc_tiling_on_sc=False)


def _gather2(tab, src, dst):
    E = src.shape[0]
    w = tab.shape[1]
    epw = E // _NW
    nch = epw // _C
    npair = (nch - 1) // 2
    assert nch == 2 * npair + 1

    @functools.partial(
        pl.kernel,
        out_type=(jax.ShapeDtypeStruct((E, w), jnp.float32),
                  jax.ShapeDtypeStruct((E, w), jnp.float32)),
        mesh=_sc_mesh(),
        compiler_params=_SC_PARAMS,
        scratch_types=[
            pltpu.VMEM((epw,), jnp.int32),
            pltpu.VMEM((epw,), jnp.int32),
            pltpu.VMEM((_C, w), jnp.float32),
            pltpu.VMEM((_C, w), jnp.float32),
            pltpu.VMEM((_C, w), jnp.float32),
            pltpu.VMEM((_C, w), jnp.float32),
            pltpu.SemaphoreType.DMA,
            pltpu.SemaphoreType.DMA,
            pltpu.SemaphoreType.DMA,
            pltpu.SemaphoreType.DMA,
        ])
    def k(t_h, src_h, dst_h, xi_h, xj_h, idd, ids,
          biA, bjA, biB, bjB, sgA, sgB, swA, swB):
        wid = lax.axis_index("s") * _NC + lax.axis_index("c")
        base0 = wid * epw
        pltpu.sync_copy(dst_h.at[pl.ds(base0, epw)], idd)
        pltpu.sync_copy(src_h.at[pl.ds(base0, epw)], ids)

        def gath(j, bi, bj, sem):
            pltpu.async_copy(t_h.at[idd.at[pl.ds(j * _C, _C)]], bi, sem)
            pltpu.async_copy(t_h.at[ids.at[pl.ds(j * _C, _C)]], bj, sem)

        def gwait(j, bi, bj, sem):
            pltpu.make_async_copy(t_h.at[idd.at[pl.ds(j * _C, _C)]],
                                  bi, sem).wait()
            pltpu.make_async_copy(t_h.at[ids.at[pl.ds(j * _C, _C)]],
                                  bj, sem).wait()

        def wback(j, bi, bj, sem):
            base = base0 + j * _C
            c1 = pltpu.async_copy(bi, xi_h.at[pl.ds(base, _C)], sem)
            c2 = pltpu.async_copy(bj, xj_h.at[pl.ds(base, _C)], sem)
            return c1, c2

        gath(0, biA, bjA, sgA)

        def pair(kk, carry):
            j0 = 2 * kk
            j1 = j0 + 1
            gath(j1, biB, bjB, sgB)
            gwait(j0, biA, bjA, sgA)
            cA1, cA2 = wback(j0, biA, bjA, swA)
            cA1.wait()
            cA2.wait()
            gath(j0 + 2, biA, bjA, sgA)
            gwait(j1, biB, bjB, sgB)
            cB1, cB2 = wback(j1, biB, bjB, swB)
            cB1.wait()
            cB2.wait()
            return carry

        lax.fori_loop(0, npair, pair, 0)
        jl = nch - 1
        gwait(jl, biA, bjA, sgA)
        cA1, cA2 = wback(jl, biA, bjA, swA)
        cA1.wait()
        cA2.wait()

    return k(tab, src, dst)


def _seg_sum(dst, n, w, vals=None, table=None, src=None):
    E = dst.shape[0]
    epw = E // _NW
    nch = epw // _C
    rps = n // _NS
    zr = 125
    nz = rps // zr
    mode = 'linear' if vals is not None else ('gather' if table is not None
                                              else 'ones')

    npair = (nch - 1) // 2
    assert nch == 2 * npair + 1
    dst2 = dst.reshape(_NW, nch, _C)

    scratch = [
        pltpu.VMEM((nch, _C), jnp.int32),
        pltpu.VMEM((epw,), jnp.int32),
        pltpu.VMEM((_C, w), jnp.float32),
        pltpu.VMEM((_C, w), jnp.float32),
        pltpu.VMEM((zr, w), jnp.float32),
        pltpu.VMEM_SHARED((n, w), jnp.float32),
        pltpu.SemaphoreType.DMA,
        pltpu.SemaphoreType.DMA,
    ]

    def body(refs):
        (v_h, dst_h, src_h, out_h, idx_d, idx_s, vbA, vbB, zb, acc,
         smA, smB) = refs
        sid = lax.axis_index("s")
        cid = lax.axis_index("c")
        wid = sid * _NC + cid
        base0 = wid * epw

        pltpu.sync_copy(dst_h.at[wid], idx_d)
        if mode == 'gather':
            pltpu.sync_copy(src_h.at[pl.ds(base0, epw)], idx_s)

        def zrow(i, c):
            for h in range(w // 16):
                zb[i, pl.ds(16 * h, 16)] = jnp.zeros((16,), jnp.float32)
            return c

        lax.fori_loop(0, zr, zrow, 0)
        for t in range(nz):
            pltpu.sync_copy(zb, acc.at[pl.ds(sid * rps + t * zr, zr)])
        if mode == 'ones':
            def orow(i, c):
                for h in range(w // 16):
                    vbA[i, pl.ds(16 * h, 16)] = jnp.ones((16,), jnp.float32)
                return c
            lax.fori_loop(0, _C, orow, 0)
        plsc.subcore_barrier()

        def load(j, vb, sem):
            if mode == 'linear':
                pltpu.async_copy(v_h.at[pl.ds(base0 + j * _C, _C)], vb, sem)
            else:
                pltpu.async_copy(v_h.at[idx_s.at[pl.ds(j * _C, _C)]], vb, sem)

        def lwait(j, vb, sem):
            if mode == 'linear':
                pltpu.make_async_copy(v_h.at[pl.ds(base0 + j * _C, _C)],
                                      vb, sem).wait()
            else:
                pltpu.make_async_copy(v_h.at[idx_s.at[pl.ds(j * _C, _C)]],
                                      vb, sem).wait()

        def scat(j, vb):
            pltpu.sync_copy(vb, acc.at[idx_d.at[j]], add=True)

        if mode == 'ones':
            def chunk(j, carry):
                scat(j, vbA)
                return carry
            lax.fori_loop(0, nch, chunk, 0)
        else:
            load(0, vbA, smA)

            def pair(kk, carry):
                j0 = 2 * kk
                j1 = j0 + 1
                load(j1, vbB, smB)
                lwait(j0, vbA, smA)
                scat(j0, vbA)
                load(j0 + 2, vbA, smA)
                lwait(j1, vbB, smB)
                scat(j1, vbB)
                return carry

            lax.fori_loop(0, npair, pair, 0)
            jl = nch - 1
            lwait(jl, vbA, smA)
            scat(jl, vbA)

        plsc.subcore_barrier()
        for t in range(nz):
            row0 = sid * rps + t * zr
            pltpu.sync_copy(acc.at[pl.ds(row0, zr)], zb)
            pltpu.sync_copy(zb, out_h.at[pl.ds(cid * n + row0, zr)])

    kern = functools.partial(
        pl.kernel,
        out_type=jax.ShapeDtypeStruct((_NC * n, w), jnp.float32),
        mesh=_sc_mesh(),
        compiler_params=_SC_PARAMS,
        scratch_types=scratch)

    if mode == 'linear':
        @kern
        def k(v_h, dst_h, out_h, *s):
            body((v_h, dst_h, None) + (out_h,) + s)
        return k(vals, dst2)
    elif mode == 'gather':
        @kern
        def k(v_h, dst_h, src_h, out_h, *s):
            body((v_h, dst_h, src_h) + (out_h,) + s)
        return k(table, dst2, src)
    else:
        @kern
        def k(dst_h, out_h, *s):
            body((None, dst_h, None) + (out_h,) + s)
        return k(dst2)


_BE = 2000


def _edge_z(XI, XJ, ci, W1, b1):
    E = XI.shape[0]
    wp = XI.shape[1]
    g = E // _BE

    def body(xi_ref, xj_ref, w_ref, b_ref, h_ref):
        xi = xi_ref[...][:, :ci]
        xj = xj_ref[...][:, :ci]
        t = jnp.concatenate([xi, xj - xi], axis=1)
        z = jnp.dot(t, w_ref[...],
                    preferred_element_type=jnp.float32) + b_ref[...]
        h_ref[...] = jnp.tanh(z)

    return pl.pallas_call(
        body,
        grid=(g,),
        in_specs=[pl.BlockSpec((_BE, wp), lambda i: (i, 0)),
                  pl.BlockSpec((_BE, wp), lambda i: (i, 0)),
                  pl.BlockSpec((2 * ci, 32), lambda i: (0, 0)),
                  pl.BlockSpec((1, 32), lambda i: (0, 0))],
        out_specs=pl.BlockSpec((_BE, 32), lambda i: (i, 0)),
        out_shape=jax.ShapeDtypeStruct((E, 32), jnp.float32),
    )(XI, XJ, W1, b1)


def _edge_mm(Hn, W, b, tanh):
    E = Hn.shape[0]
    wp = W.shape[1]
    g = E // _BE

    def body(h_ref, w_ref, b_ref, o_ref):
        z = jnp.dot(h_ref[...], w_ref[...],
                    preferred_element_type=jnp.float32) + b_ref[...]
        o_ref[...] = jnp.tanh(z) if tanh else z

    return pl.pallas_call(
        body,
        grid=(g,),
        in_specs=[pl.BlockSpec((_BE, 32), lambda i: (i, 0)),
                  pl.BlockSpec((32, wp), lambda i: (0, 0)),
                  pl.BlockSpec((1, wp), lambda i: (0, 0))],
        out_specs=pl.BlockSpec((_BE, wp), lambda i: (i, 0)),
        out_shape=jax.ShapeDtypeStruct((E, wp), jnp.float32),
    )(Hn, W, b)


def _node(X1, W1, X2=None, W2=None, b=None, T=None, leaky=False):
    n = X1.shape[0]
    co = W1.shape[1]
    bn = 2000
    g = n // bn

    has2 = X2 is not None
    hasT = T is not None
    hasb = b is not None

    def body(*refs):
        it = iter(refs)
        x1 = next(it)[...]
        w1 = next(it)[...]
        acc = jnp.dot(x1, w1, preferred_element_type=jnp.float32)
        if has2:
            acc += jnp.dot(next(it)[...], next(it)[...],
                           preferred_element_type=jnp.float32)
        if hasT:
            acc += next(it)[...]
        if hasb:
            acc += next(it)[...]
        o = next(it)
        if leaky:
            acc = jnp.where(acc >= 0, acc, 0.1 * acc)
        o[...] = acc

    c1 = X1.shape[1]
    in_specs = [pl.BlockSpec((bn, c1), lambda i: (i, 0)),
                pl.BlockSpec((c1, co), lambda i: (0, 0))]
    args = [X1, W1]
    if has2:
        c2 = X2.shape[1]
        in_specs += [pl.BlockSpec((bn, c2), lambda i: (i, 0)),
                     pl.BlockSpec((c2, co), lambda i: (0, 0))]
        args += [X2, W2]
    if hasT:
        in_specs.append(pl.BlockSpec((bn, co), lambda i: (i, 0)))
        args.append(T)
    if hasb:
        in_specs.append(pl.BlockSpec((1, co), lambda i: (0, 0)))
        args.append(b)

    return pl.pallas_call(
        body,
        grid=(g,),
        in_specs=in_specs,
        out_specs=pl.BlockSpec((bn, co), lambda i: (i, 0)),
        out_shape=jax.ShapeDtypeStruct((n, co), jnp.float32),
    )(*args)


def _padw(a, w):
    return a if a.shape[1] == w else jnp.pad(a, ((0, 0), (0, w - a.shape[1])))


def kernel(x, pos, edge_index, batch, params):
    src = edge_index[0].astype(jnp.int32)
    dst = edge_index[1].astype(jnp.int32)
    n = x.shape[0]
    E = src.shape[0]

    cntP = _seg_sum(dst, n, 16)
    cnt = cntP[:n, 0] + cntP[n:, 0]
    cmax = jnp.maximum(cnt, 1.0)

    def _bn(h, g, b):
        m = jnp.mean(h, axis=0)
        v = jnp.var(h, axis=0)
        return (h - m) / jnp.sqrt(v + _EPS) * g + b

    def ec_layer(p, xin):
        ci = xin.shape[1]
        wp = max(16, ci)
        XI, XJ = _gather2(_padw(xin, wp), src, dst)
        H1 = _edge_z(XI, XJ, ci, p['W1'], p['b1'][None])
        H1n = _bn(H1, p['g1'], p['be1'])
        H2 = _edge_mm(H1n, p['W2'], p['b2'][None], tanh=True)
        H2n = _bn(H2, p['g2'], p['be2'])
        cop = max(16, p['W3'].shape[1])
        Y = _edge_mm(H2n, _padw(p['W3'], cop), _padw(p['b3'][None], cop),
                     tanh=False)
        P = _seg_sum(dst, n, cop, vals=Y)
        return (P[:n] + P[n:])[:, :p['W3'].shape[1]]

    h = ec_layer(params['ec1'], pos)
    app = h
    for nm in ['eca', 'ecb', 'ecc']:
        h = ec_layer(params[nm], h)
        app = jnp.concatenate([app, h], axis=1)
    h = ec_layer(params['ec2'], app)
    h = ec_layer(params['ec3'], h)

    def sage(p, u, leaky):
        ci = u.shape[1]
        w = max(16, ci)
        S = _seg_sum(dst, n, w, table=_padw(u, w), src=src)
        mean = (S[:n, :ci] + S[n:, :ci]) / cmax[:, None]
        return _node(mean, p['Wl'], X2=u, W2=p['Wr'], b=p['b'][None],
                     leaky=leaky)

    u = sage(params['sg4'], x + h, True)
    for nm in ['sga', 'sgb', 'sgc']:
        u = sage(params[nm], u, True)
    return sage(params['sg5'], u, False)


# speedup vs baseline: 2.0453x; 1.0643x over previous
import functools

import jax
import jax.numpy as jnp
from jax import lax
from jax.experimental import pallas as pl
from jax.experimental.pallas import tpu as pltpu
from jax.experimental.pallas import tpu_sc as plsc

_NC, _NS = 2, 16
_NW = _NC * _NS
_C = 80
_EPS = 1e-5


def _sc_mesh():
    return plsc.VectorSubcoreMesh(
        core_axis_name="c", subcore_axis_name="s",
        num_cores=_NC, num_subcores=_NS)


_SC_PARAMS = pltpu.CompilerParams(use_tc_tiling_on_sc=False)


def _gather2(tab, src, dst):
    E = src.shape[0]
    w = tab.shape[1]
    epw = E // _NW
    nch = epw // _C
    npair = (nch - 1) // 2
    assert nch == 2 * npair + 1

    @functools.partial(
        pl.kernel,
        out_type=(jax.ShapeDtypeStruct((E, w), jnp.float32),
                  jax.ShapeDtypeStruct((E, w), jnp.float32)),
        mesh=_sc_mesh(),
        compiler_params=_SC_PARAMS,
        scratch_types=[
            pltpu.VMEM((epw,), jnp.int32),
            pltpu.VMEM((epw,), jnp.int32),
            pltpu.VMEM((_C, w), jnp.float32),
            pltpu.VMEM((_C, w), jnp.float32),
            pltpu.VMEM((_C, w), jnp.float32),
            pltpu.VMEM((_C, w), jnp.float32),
            pltpu.SemaphoreType.DMA,
            pltpu.SemaphoreType.DMA,
            pltpu.SemaphoreType.DMA,
            pltpu.SemaphoreType.DMA,
        ])
    def k(t_h, src_h, dst_h, xi_h, xj_h, idd, ids,
          biA, bjA, biB, bjB, sgA, sgB, swA, swB):
        wid = lax.axis_index("s") * _NC + lax.axis_index("c")
        base0 = wid * epw
        pltpu.sync_copy(dst_h.at[pl.ds(base0, epw)], idd)
        pltpu.sync_copy(src_h.at[pl.ds(base0, epw)], ids)

        def gath(j, bi, bj, sem):
            pltpu.async_copy(t_h.at[idd.at[pl.ds(j * _C, _C)]], bi, sem)
            pltpu.async_copy(t_h.at[ids.at[pl.ds(j * _C, _C)]], bj, sem)

        def gwait(j, bi, bj, sem):
            pltpu.make_async_copy(t_h.at[idd.at[pl.ds(j * _C, _C)]],
                                  bi, sem).wait()
            pltpu.make_async_copy(t_h.at[ids.at[pl.ds(j * _C, _C)]],
                                  bj, sem).wait()

        def wback(j, bi, bj, sem):
            base = base0 + j * _C
            c1 = pltpu.async_copy(bi, xi_h.at[pl.ds(base, _C)], sem)
            c2 = pltpu.async_copy(bj, xj_h.at[pl.ds(base, _C)], sem)
            return c1, c2

        gath(0, biA, bjA, sgA)

        def pair(kk, carry):
            j0 = 2 * kk
            j1 = j0 + 1
            gath(j1, biB, bjB, sgB)
            gwait(j0, biA, bjA, sgA)
            cA1, cA2 = wback(j0, biA, bjA, swA)
            cA1.wait()
            cA2.wait()
            gath(j0 + 2, biA, bjA, sgA)
            gwait(j1, biB, bjB, sgB)
            cB1, cB2 = wback(j1, biB, bjB, swB)
            cB1.wait()
            cB2.wait()
            return carry

        lax.fori_loop(0, npair, pair, 0)
        jl = nch - 1
        gwait(jl, biA, bjA, sgA)
        cA1, cA2 = wback(jl, biA, bjA, swA)
        cA1.wait()
        cA2.wait()

    return k(tab, src, dst)


def _seg_sum(dst, n, w, vals=None, table=None, src=None):
    E = dst.shape[0]
    epw = E // _NW
    nch = epw // _C
    rps = n // _NS
    zr = 125
    nz = rps // zr
    mode = 'linear' if vals is not None else ('gather' if table is not None
                                              else 'ones')

    npair = (nch - 1) // 2
    assert nch == 2 * npair + 1
    dst2 = dst.reshape(_NW, nch, _C)

    scratch = [
        pltpu.VMEM((nch, _C), jnp.int32),
        pltpu.VMEM((epw,), jnp.int32),
        pltpu.VMEM((_C, w), jnp.float32),
        pltpu.VMEM((_C, w), jnp.float32),
        pltpu.VMEM((zr, w), jnp.float32),
        pltpu.VMEM_SHARED((n, w), jnp.float32),
        pltpu.SemaphoreType.DMA,
        pltpu.SemaphoreType.DMA,
    ]

    def body(refs):
        (v_h, dst_h, src_h, out_h, idx_d, idx_s, vbA, vbB, zb, acc,
         smA, smB) = refs
        sid = lax.axis_index("s")
        cid = lax.axis_index("c")
        wid = sid * _NC + cid
        base0 = wid * epw

        pltpu.sync_copy(dst_h.at[wid], idx_d)
        if mode == 'gather':
            pltpu.sync_copy(src_h.at[pl.ds(base0, epw)], idx_s)

        def zrow(i, c):
            for h in range(w // 16):
                zb[i, pl.ds(16 * h, 16)] = jnp.zeros((16,), jnp.float32)
            return c

        lax.fori_loop(0, zr, zrow, 0)
        for t in range(nz):
            pltpu.sync_copy(zb, acc.at[pl.ds(sid * rps + t * zr, zr)])
        if mode == 'ones':
            def orow(i, c):
                for h in range(w // 16):
                    vbA[i, pl.ds(16 * h, 16)] = jnp.ones((16,), jnp.float32)
                return c
            lax.fori_loop(0, _C, orow, 0)
        plsc.subcore_barrier()

        def load(j, vb, sem):
            if mode == 'linear':
                pltpu.async_copy(v_h.at[pl.ds(base0 + j * _C, _C)], vb, sem)
            else:
                pltpu.async_copy(v_h.at[idx_s.at[pl.ds(j * _C, _C)]], vb, sem)

        def lwait(j, vb, sem):
            if mode == 'linear':
                pltpu.make_async_copy(v_h.at[pl.ds(base0 + j * _C, _C)],
                                      vb, sem).wait()
            else:
                pltpu.make_async_copy(v_h.at[idx_s.at[pl.ds(j * _C, _C)]],
                                      vb, sem).wait()

        def scat(j, vb):
            pltpu.sync_copy(vb, acc.at[idx_d.at[j]], add=True)

        if mode == 'ones':
            def chunk(j, carry):
                scat(j, vbA)
                return carry
            lax.fori_loop(0, nch, chunk, 0)
        else:
            load(0, vbA, smA)

            def pair(kk, carry):
                j0 = 2 * kk
                j1 = j0 + 1
                load(j1, vbB, smB)
                lwait(j0, vbA, smA)
                scat(j0, vbA)
                load(j0 + 2, vbA, smA)
                lwait(j1, vbB, smB)
                scat(j1, vbB)
                return carry

            lax.fori_loop(0, npair, pair, 0)
            jl = nch - 1
            lwait(jl, vbA, smA)
            scat(jl, vbA)

        plsc.subcore_barrier()
        for t in range(nz):
            row0 = sid * rps + t * zr
            pltpu.sync_copy(acc.at[pl.ds(row0, zr)], zb)
            pltpu.sync_copy(zb, out_h.at[pl.ds(cid * n + row0, zr)])

    kern = functools.partial(
        pl.kernel,
        out_type=jax.ShapeDtypeStruct((_NC * n, w), jnp.float32),
        mesh=_sc_mesh(),
        compiler_params=_SC_PARAMS,
        scratch_types=scratch)

    if mode == 'linear':
        @kern
        def k(v_h, dst_h, out_h, *s):
            body((v_h, dst_h, None) + (out_h,) + s)
        return k(vals, dst2)
    elif mode == 'gather':
        @kern
        def k(v_h, dst_h, src_h, out_h, *s):
            body((v_h, dst_h, src_h) + (out_h,) + s)
        return k(table, dst2, src)
    else:
        @kern
        def k(dst_h, out_h, *s):
            body((None, dst_h, None) + (out_h,) + s)
        return k(dst2)


_BE = 2000


def _edge_z(XI, XJ, ci, W1, b1):
    E = XI.shape[0]
    wp = XI.shape[1]
    g = E // _BE

    def body(xi_ref, xj_ref, w_ref, b_ref, h_ref):
        xi = xi_ref[...][:, :ci]
        xj = xj_ref[...][:, :ci]
        t = jnp.concatenate([xi, xj - xi], axis=1)
        z = jnp.dot(t, w_ref[...],
                    preferred_element_type=jnp.float32) + b_ref[...]
        h_ref[...] = jnp.tanh(z)

    return pl.pallas_call(
        body,
        grid=(g,),
        in_specs=[pl.BlockSpec((_BE, wp), lambda i: (i, 0)),
                  pl.BlockSpec((_BE, wp), lambda i: (i, 0)),
                  pl.BlockSpec((2 * ci, 32), lambda i: (0, 0)),
                  pl.BlockSpec((1, 32), lambda i: (0, 0))],
        out_specs=pl.BlockSpec((_BE, 32), lambda i: (i, 0)),
        out_shape=jax.ShapeDtypeStruct((E, 32), jnp.float32),
    )(XI, XJ, W1, b1)


def _edge_z4(pairs, W1, b1):
    E = pairs[0][0].shape[0]
    g = E // _BE

    def body(*refs):
        xis = [refs[i][...] for i in range(4)]
        xjs = [refs[4 + i][...] for i in range(4)]
        w_ref, b_ref, h_ref = refs[8], refs[9], refs[10]
        t = jnp.concatenate(xis + [xj - xi for xi, xj in zip(xis, xjs)],
                            axis=1)
        z = jnp.dot(t, w_ref[...],
                    preferred_element_type=jnp.float32) + b_ref[...]
        h_ref[...] = jnp.tanh(z)

    blk = lambda: pl.BlockSpec((_BE, 64), lambda i: (i, 0))
    return pl.pallas_call(
        body,
        grid=(g,),
        in_specs=[blk() for _ in range(8)]
                 + [pl.BlockSpec((512, 32), lambda i: (0, 0)),
                    pl.BlockSpec((1, 32), lambda i: (0, 0))],
        out_specs=pl.BlockSpec((_BE, 32), lambda i: (i, 0)),
        out_shape=jax.ShapeDtypeStruct((E, 32), jnp.float32),
    )(*[p[0] for p in pairs], *[p[1] for p in pairs], W1, b1)


def _edge_mm(Hn, W, b, tanh):
    E = Hn.shape[0]
    wp = W.shape[1]
    g = E // _BE

    def body(h_ref, w_ref, b_ref, o_ref):
        z = jnp.dot(h_ref[...], w_ref[...],
                    preferred_element_type=jnp.float32) + b_ref[...]
        o_ref[...] = jnp.tanh(z) if tanh else z

    return pl.pallas_call(
        body,
        grid=(g,),
        in_specs=[pl.BlockSpec((_BE, 32), lambda i: (i, 0)),
                  pl.BlockSpec((32, wp), lambda i: (0, 0)),
                  pl.BlockSpec((1, wp), lambda i: (0, 0))],
        out_specs=pl.BlockSpec((_BE, wp), lambda i: (i, 0)),
        out_shape=jax.ShapeDtypeStruct((E, wp), jnp.float32),
    )(Hn, W, b)


def _node(X1, W1, X2=None, W2=None, b=None, T=None, leaky=False):
    n = X1.shape[0]
    co = W1.shape[1]
    bn = 2000
    g = n // bn

    has2 = X2 is not None
    hasT = T is not None
    hasb = b is not None

    def body(*refs):
        it = iter(refs)
        x1 = next(it)[...]
        w1 = next(it)[...]
        acc = jnp.dot(x1, w1, preferred_element_type=jnp.float32)
        if has2:
            acc += jnp.dot(next(it)[...], next(it)[...],
                           preferred_element_type=jnp.float32)
        if hasT:
            acc += next(it)[...]
        if hasb:
            acc += next(it)[...]
        o = next(it)
        if leaky:
            acc = jnp.where(acc >= 0, acc, 0.1 * acc)
        o[...] = acc

    c1 = X1.shape[1]
    in_specs = [pl.BlockSpec((bn, c1), lambda i: (i, 0)),
                pl.BlockSpec((c1, co), lambda i: (0, 0))]
    args = [X1, W1]
    if has2:
        c2 = X2.shape[1]
        in_specs += [pl.BlockSpec((bn, c2), lambda i: (i, 0)),
                     pl.BlockSpec((c2, co), lambda i: (0, 0))]
        args += [X2, W2]
    if hasT:
        in_specs.append(pl.BlockSpec((bn, co), lambda i: (i, 0)))
        args.append(T)
    if hasb:
        in_specs.append(pl.BlockSpec((1, co), lambda i: (0, 0)))
        args.append(b)

    return pl.pallas_call(
        body,
        grid=(g,),
        in_specs=in_specs,
        out_specs=pl.BlockSpec((bn, co), lambda i: (i, 0)),
        out_shape=jax.ShapeDtypeStruct((n, co), jnp.float32),
    )(*args)


def _padw(a, w):
    return a if a.shape[1] == w else jnp.pad(a, ((0, 0), (0, w - a.shape[1])))


def kernel(x, pos, edge_index, batch, params):
    src = edge_index[0].astype(jnp.int32)
    dst = edge_index[1].astype(jnp.int32)
    n = x.shape[0]
    E = src.shape[0]

    cntP = _seg_sum(dst, n, 16)
    cnt = cntP[:n, 0] + cntP[n:, 0]
    cmax = jnp.maximum(cnt, 1.0)

    def _bn(h, g, b):
        m = jnp.mean(h, axis=0)
        v = jnp.var(h, axis=0)
        return (h - m) / jnp.sqrt(v + _EPS) * g + b

    def ec_rest(p, H1):
        H1n = _bn(H1, p['g1'], p['be1'])
        H2 = _edge_mm(H1n, p['W2'], p['b2'][None], tanh=True)
        H2n = _bn(H2, p['g2'], p['be2'])
        cop = max(16, p['W3'].shape[1])
        Y = _edge_mm(H2n, _padw(p['W3'], cop), _padw(p['b3'][None], cop),
                     tanh=False)
        P = _seg_sum(dst, n, cop, vals=Y)
        return (P[:n] + P[n:])[:, :p['W3'].shape[1]]

    def ec_layer(p, xin):
        ci = xin.shape[1]
        wp = max(16, ci)
        XI, XJ = _gather2(_padw(xin, wp), src, dst)
        H1 = _edge_z(XI, XJ, ci, p['W1'], p['b1'][None])
        return ec_rest(p, H1), (XI, XJ)

    h, _ = ec_layer(params['ec1'], pos)
    pairs = []
    for nm in ['eca', 'ecb', 'ecc']:
        h, pair = ec_layer(params[nm], h)
        pairs.append(pair)
    pairs.append(_gather2(h, src, dst))
    H1 = _edge_z4(pairs, params['ec2']['W1'], params['ec2']['b1'][None])
    h = ec_rest(params['ec2'], H1)
    h, _ = ec_layer(params['ec3'], h)

    def sage(p, u, leaky):
        ci = u.shape[1]
        w = max(16, ci)
        S = _seg_sum(dst, n, w, table=_padw(u, w), src=src)
        mean = (S[:n, :ci] + S[n:, :ci]) / cmax[:, None]
        return _node(mean, p['Wl'], X2=u, W2=p['Wr'], b=p['b'][None],
                     leaky=leaky)

    u = sage(params['sg4'], x + h, True)
    for nm in ['sga', 'sgb', 'sgc']:
        u = sage(params[nm], u, True)
    return sage(params['sg5'], u, False)
